# baseline (speedup 1.0000x reference)
import functools

import jax
import jax.numpy as jnp
from jax import lax
from jax.experimental import pallas as pl
from jax.experimental.pallas import tpu as pltpu
from jax.experimental.pallas import tpu_sc as plsc

_L = 1000
_CHUNK_ROWS = 64
_IDX_MINOR = 128


@functools.partial(jax.jit, static_argnums=(1,))
def _one_hot_sc(x_flat, rows):
    info = plsc.get_sparse_core_info()
    nw = info.num_cores * info.num_subcores
    rpw = rows // nw
    n_chunks = rpw // _CHUNK_ROWS
    n_idx_rows = rpw // _IDX_MINOR
    chunk_words = _CHUNK_ROWS * _L
    mesh = plsc.VectorSubcoreMesh(core_axis_name="c", subcore_axis_name="s")

    @functools.partial(
        pl.kernel,
        out_type=jax.ShapeDtypeStruct((rows * _L,), jnp.float32),
        mesh=mesh,
        scratch_types=[
            pltpu.VMEM((rpw,), jnp.int32),
            pltpu.VMEM((chunk_words,), jnp.float32),
            pltpu.VMEM((n_idx_rows, _IDX_MINOR), jnp.int32),
            pltpu.VMEM((n_idx_rows, _IDX_MINOR), jnp.float32),
            pltpu.SemaphoreType.DMA,
            pltpu.SemaphoreType.DMA,
        ],
    )
    def k(x_hbm, out_hbm, xbuf, zbuf, idx, ones, sem_z, sem_o):
        wid = lax.axis_index("s") * info.num_cores + lax.axis_index("c")
        base_row = wid * rpw
        pltpu.sync_copy(x_hbm.at[pl.ds(base_row, rpw)], xbuf)

        zero16 = jnp.zeros((16,), jnp.float32)
        one16 = jnp.ones((16,), jnp.float32)
        iota16 = lax.iota(jnp.int32, 16)

        def zero_fill(i, carry):
            for kk in range(16):
                zbuf[pl.ds(i * 256 + kk * 16, 16)] = zero16
            return carry

        lax.fori_loop(0, chunk_words // 256, zero_fill, 0)

        def build_idx(j, carry):
            for kk in range(_IDX_MINOR // 16):
                off = j * _IDX_MINOR + kk * 16
                xv = xbuf[pl.ds(off, 16)]
                row_ids = base_row + off + iota16
                idx[j, pl.ds(kk * 16, 16)] = row_ids * _L + xv
                ones[j, pl.ds(kk * 16, 16)] = one16
            return carry

        lax.fori_loop(0, n_idx_rows, build_idx, 0)

        out_base = base_row * _L
        zcopies = [
            pltpu.async_copy(
                zbuf,
                out_hbm.at[pl.ds(out_base + g * chunk_words, chunk_words)],
                sem_z,
            )
            for g in range(n_chunks)
        ]
        for c in zcopies:
            c.wait()

        ocopies = [
            pltpu.async_copy(ones.at[j], out_hbm.at[idx.at[j]], sem_o)
            for j in range(n_idx_rows)
        ]
        for c in ocopies:
            c.wait()

    return k(x_flat)


def kernel(x):
    b, f = x.shape
    rows = b * f
    out_flat = _one_hot_sc(x.reshape(rows), rows)
    return out_flat.reshape(b, f, _L)


# speedup vs baseline: 8.0833x; 8.0833x over previous
import functools

import jax
import jax.numpy as jnp
from jax import lax
from jax.experimental import pallas as pl
from jax.experimental.pallas import tpu as pltpu
from jax.experimental.pallas import tpu_sc as plsc

_L = 1000
_BT = 128
_HALVES = ((0, 512), (512, 488))
_XRING = 4
_NUM_CORES = 2
_NUM_SUBCORES = 16


@functools.partial(jax.jit, static_argnums=(1, 2))
def _one_hot_sc_t(xt, b, f):
    nw = _NUM_CORES * _NUM_SUBCORES
    nbt = b // _BT
    spw = f * nbt // nw
    mesh = plsc.VectorSubcoreMesh(core_axis_name="c", subcore_axis_name="s",
                                  num_cores=_NUM_CORES,
                                  num_subcores=_NUM_SUBCORES)

    @functools.partial(
        pl.kernel,
        out_type=jax.ShapeDtypeStruct((f, _L, b), jnp.float32),
        mesh=mesh,
        compiler_params=pltpu.CompilerParams(needs_layout_passes=False),
        scratch_types=[
            pltpu.VMEM((_XRING, _BT), jnp.int32),
            pltpu.VMEM((_L, _BT), jnp.float32),
            pltpu.SemaphoreType.DMA,
            pltpu.SemaphoreType.DMA,
            pltpu.SemaphoreType.DMA,
        ],
    )
    def k(xt_hbm, out_hbm, xbuf, slab, sem_x, sem_h0, sem_h1):
        wid = lax.axis_index("s") * _NUM_CORES + lax.axis_index("c")

        zero16 = jnp.zeros((16,), jnp.float32)
        one16 = jnp.ones((16,), jnp.float32)
        iota16 = lax.iota(jnp.int32, 16)

        fis, bts = [], []
        for j in range(spw):
            sid = wid * spw + j
            fis.append(sid // nbt)
            bts.append((sid % nbt) * _BT)

        def fetch_x(j):
            return pltpu.async_copy(
                xt_hbm.at[fis[j], pl.ds(bts[j], _BT)],
                xbuf.at[j % _XRING],
                sem_x,
            )

        xcopies = [fetch_x(j) for j in range(_XRING - 1)]

        def zero_fill(l, carry):
            for c in range(_BT // 16):
                slab[l, pl.ds(c * 16, 16)] = zero16
            return carry

        lax.fori_loop(0, _L, zero_fill, 0)

        def touch(j, l0, hlen, val16):
            for c in range(_BT // 16):
                xv = xbuf[j % _XRING, pl.ds(c * 16, 16)]
                m = (xv >= l0) & (xv < l0 + hlen)
                plsc.store_scatter(slab, [xv, iota16 + c * 16], val16, mask=m)

        sems = (sem_h0, sem_h1)
        pend = [None, None]
        prev = [None, None]
        for j in range(spw):
            xcopies[j].wait()
            for h, (l0, hlen) in enumerate(_HALVES):
                if pend[h] is not None:
                    pend[h].wait()
                    touch(prev[h], l0, hlen, zero16)
                touch(j, l0, hlen, one16)
                pend[h] = pltpu.async_copy(
                    slab.at[pl.ds(l0, hlen)],
                    out_hbm.at[fis[j], pl.ds(l0, hlen), pl.ds(bts[j], _BT)],
                    sems[h],
                )
                prev[h] = j
            if j + _XRING - 1 < spw:
                xcopies.append(fetch_x(j + _XRING - 1))
        for p in pend:
            p.wait()

    return k(xt)


def kernel(x):
    b, f = x.shape
    out_t = _one_hot_sc_t(x.T, b, f)
    return out_t.transpose(2, 0, 1)


# speedup vs baseline: 8.1316x; 1.0060x over previous
import functools

import jax
import jax.numpy as jnp
from jax import lax
from jax.experimental import pallas as pl
from jax.experimental.pallas import tpu as pltpu
from jax.experimental.pallas import tpu_sc as plsc

_L = 1000
_BT = 128
_HALVES = ((0, 512), (512, 488))
_XRING = 4
_NUM_CORES = 2
_NUM_SUBCORES = 16


@functools.partial(jax.jit, static_argnums=(1, 2))
def _one_hot_sc_t(xt, b, f):
    nw = _NUM_CORES * _NUM_SUBCORES
    nbt = b // _BT
    spw = f * nbt // nw
    mesh = plsc.VectorSubcoreMesh(core_axis_name="c", subcore_axis_name="s",
                                  num_cores=_NUM_CORES,
                                  num_subcores=_NUM_SUBCORES)

    @functools.partial(
        pl.kernel,
        out_type=jax.ShapeDtypeStruct((f, _L, b), jnp.float32),
        mesh=mesh,
        compiler_params=pltpu.CompilerParams(needs_layout_passes=False),
        scratch_types=[
            pltpu.VMEM((_XRING, _BT), jnp.int32),
            pltpu.VMEM((_L, _BT), jnp.float32),
            pltpu.SemaphoreType.DMA,
            pltpu.SemaphoreType.DMA,
            pltpu.SemaphoreType.DMA,
        ],
    )
    def k(xt_hbm, out_hbm, xbuf, slab, sem_x, sem_h0, sem_h1):
        wid = lax.axis_index("s") * _NUM_CORES + lax.axis_index("c")

        zero16 = jnp.zeros((16,), jnp.float32)
        one16 = jnp.ones((16,), jnp.float32)
        iota16 = lax.iota(jnp.int32, 16)

        fis, bts = [], []
        for j in range(spw):
            sid = wid * spw + j
            fis.append(sid // nbt)
            bts.append((sid % nbt) * _BT)

        def fetch_x(j):
            return pltpu.async_copy(
                xt_hbm.at[fis[j], pl.ds(bts[j], _BT)],
                xbuf.at[j % _XRING],
                sem_x,
            )

        xcopies = [fetch_x(j) for j in range(_XRING - 1)]

        def zero_fill(i, carry):
            for r in range(8):
                for c in range(_BT // 16):
                    slab[i * 8 + r, pl.ds(c * 16, 16)] = zero16
            return carry

        def touch(j, l0, hlen, val16):
            for c in range(_BT // 16):
                xv = xbuf[j % _XRING, pl.ds(c * 16, 16)]
                m = (xv >= l0) & (xv < l0 + hlen)
                plsc.store_scatter(slab, [xv, iota16 + c * 16], val16, mask=m)

        sems = (sem_h0, sem_h1)
        pend = [None, None]
        prev = [None, None]
        for j in range(spw):
            xcopies[j].wait()
            for h, (l0, hlen) in enumerate(_HALVES):
                if j == 0:
                    lax.fori_loop(l0 // 8, (l0 + hlen) // 8, zero_fill, 0)
                if pend[h] is not None:
                    pend[h].wait()
                    touch(prev[h], l0, hlen, zero16)
                touch(j, l0, hlen, one16)
                pend[h] = pltpu.async_copy(
                    slab.at[pl.ds(l0, hlen)],
                    out_hbm.at[fis[j], pl.ds(l0, hlen), pl.ds(bts[j], _BT)],
                    sems[h],
                )
                prev[h] = j
            if j + _XRING - 1 < spw:
                xcopies.append(fetch_x(j + _XRING - 1))
        for p in pend:
            p.wait()

    return k(xt)


def kernel(x):
    b, f = x.shape
    out_t = _one_hot_sc_t(x.T, b, f)
    return out_t.transpose(2, 0, 1)


# speedup vs baseline: 8.5451x; 1.0509x over previous
import functools

import jax
import jax.numpy as jnp
from jax import lax
from jax.experimental import pallas as pl
from jax.experimental.pallas import tpu as pltpu
from jax.experimental.pallas import tpu_sc as plsc

_L = 1000
_BT = 128
_PIECES = ((0, 256), (256, 256), (512, 256), (768, 232))
_XRING = 4
_NUM_CORES = 2
_NUM_SUBCORES = 16


@functools.partial(jax.jit, static_argnums=(1, 2))
def _one_hot_sc_t(xt, b, f):
    nw = _NUM_CORES * _NUM_SUBCORES
    nbt = b // _BT
    spw = f * nbt // nw
    mesh = plsc.VectorSubcoreMesh(core_axis_name="c", subcore_axis_name="s",
                                  num_cores=_NUM_CORES,
                                  num_subcores=_NUM_SUBCORES)

    @functools.partial(
        pl.kernel,
        out_type=jax.ShapeDtypeStruct((f, _L, b), jnp.float32),
        mesh=mesh,
        compiler_params=pltpu.CompilerParams(needs_layout_passes=False),
        scratch_types=[
            pltpu.VMEM((_XRING, _BT), jnp.int32),
            pltpu.VMEM((_L, _BT), jnp.float32),
            pltpu.SemaphoreType.DMA,
            pltpu.SemaphoreType.DMA,
            pltpu.SemaphoreType.DMA,
            pltpu.SemaphoreType.DMA,
            pltpu.SemaphoreType.DMA,
        ],
    )
    def k(xt_hbm, out_hbm, xbuf, slab, sem_x, sem_p0, sem_p1, sem_p2, sem_p3):
        wid = lax.axis_index("s") * _NUM_CORES + lax.axis_index("c")
        base_sid = wid * spw
        sems = (sem_p0, sem_p1, sem_p2, sem_p3)

        zero16 = jnp.zeros((16,), jnp.float32)
        one16 = jnp.ones((16,), jnp.float32)
        iota16 = lax.iota(jnp.int32, 16)

        def fetch_x(j):
            sid = base_sid + j
            pltpu.async_copy(
                xt_hbm.at[sid // nbt, pl.ds((sid % nbt) * _BT, _BT)],
                xbuf.at[j % _XRING],
                sem_x,
            )

        def drain_x():
            pltpu.make_async_copy(
                xt_hbm.at[0, pl.ds(0, _BT)], xbuf.at[0], sem_x
            ).wait()

        def drain_piece(h):
            l0, hlen = _PIECES[h]
            pltpu.make_async_copy(
                slab.at[pl.ds(l0, hlen)],
                out_hbm.at[0, pl.ds(l0, hlen), pl.ds(0, _BT)],
                sems[h],
            ).wait()

        def fire_piece(h, j):
            l0, hlen = _PIECES[h]
            sid = base_sid + j
            pltpu.async_copy(
                slab.at[pl.ds(l0, hlen)],
                out_hbm.at[
                    sid // nbt, pl.ds(l0, hlen), pl.ds((sid % nbt) * _BT, _BT)
                ],
                sems[h],
            )

        def touch(j, l0, hlen, val16):
            for c in range(_BT // 16):
                xv = xbuf[j % _XRING, pl.ds(c * 16, 16)]
                m = (xv >= l0) & (xv < l0 + hlen)
                plsc.store_scatter(slab, [xv, iota16 + c * 16], val16, mask=m)

        def zero_rows(i, carry):
            for r in range(8):
                for c in range(_BT // 16):
                    slab[i * 8 + r, pl.ds(c * 16, 16)] = zero16
            return carry

        for j in range(_XRING - 1):
            fetch_x(j)
        drain_x()
        for h, (l0, hlen) in enumerate(_PIECES):
            lax.fori_loop(l0 // 8, (l0 + hlen) // 8, zero_rows, 0)
            touch(0, l0, hlen, one16)
            fire_piece(h, 0)
        fetch_x(_XRING - 1)

        def body(j, carry):
            drain_x()
            for h, (l0, hlen) in enumerate(_PIECES):
                drain_piece(h)
                touch(j - 1, l0, hlen, zero16)
                touch(j, l0, hlen, one16)
                fire_piece(h, j)

            @pl.when(j + _XRING - 1 < spw)
            def _():
                fetch_x(j + _XRING - 1)

            return carry

        lax.fori_loop(1, spw, body, 0)
        for h in range(len(_PIECES)):
            drain_piece(h)

    return k(xt)


def kernel(x):
    b, f = x.shape
    out_t = _one_hot_sc_t(x.T, b, f)
    return out_t.transpose(2, 0, 1)
